# baseline (speedup 1.0000x reference)
import functools

import jax
import jax.numpy as jnp
from jax.experimental import pallas as pl
from jax.experimental.pallas import tpu as pltpu

EPS = 1e-5
SLOPE = 0.01
NCLS = 9
VMEM_LIMIT = 64 * 1024 * 1024


def _lrelu(y):
    return jnp.where(y >= 0.0, y, SLOPE * y)


def _largest_divisor_leq(v, cap):
    for d in range(min(cap, v), 0, -1):
        if v % d == 0:
            return d
    return 1


def _shift_gemm(x, w_ref, b_ref, *, ow, m, kh):
    z = None
    for a in range(kh):
        pa = x[a * ow: a * ow + m, :]
        d = jnp.dot(pa, w_ref[a], preferred_element_type=jnp.float32)
        z = d if z is None else z + d
    return z + b_ref[...]


def _conv1_stats_kernel(x_ref, w_ref, b_ref, s_ref, q_ref, *, oh, ow, kh):
    z = _shift_gemm(x_ref[0], w_ref, b_ref, ow=ow, m=oh * ow, kh=kh)
    s_ref[0] = jnp.sum(z, axis=0, keepdims=True)
    q_ref[0] = jnp.sum(z * z, axis=0, keepdims=True)


def _conv1_stats(xr, w1r, b1, *, oh, ow, kh, kwid, cout):
    n, rows, _ = xr.shape
    s, q = pl.pallas_call(
        functools.partial(_conv1_stats_kernel, oh=oh, ow=ow, kh=kh),
        out_shape=(jax.ShapeDtypeStruct((n, 1, cout), jnp.float32),
                   jax.ShapeDtypeStruct((n, 1, cout), jnp.float32)),
        grid=(n,),
        in_specs=[pl.BlockSpec((1, rows, kwid), lambda i: (i, 0, 0)),
                  pl.BlockSpec((kh, kwid, cout), lambda i: (0, 0, 0)),
                  pl.BlockSpec((1, cout), lambda i: (0, 0))],
        out_specs=(pl.BlockSpec((1, 1, cout), lambda i: (i, 0, 0)),
                   pl.BlockSpec((1, 1, cout), lambda i: (i, 0, 0))),
        compiler_params=pltpu.CompilerParams(
            dimension_semantics=("parallel",),
            vmem_limit_bytes=VMEM_LIMIT),
    )(xr, w1r, b1)
    return s, q


def _conv1_pool_kernel(x_ref, w_ref, b_ref, sc_ref, sh_ref, o_ref,
                       *, oh, ow, kh):
    z = _shift_gemm(x_ref[0], w_ref, b_ref, ow=ow, m=oh * ow, kh=kh)
    y = _lrelu(z * sc_ref[...] + sh_ref[...])
    v = y.reshape(oh // 2, 2, ow // 2, 2, y.shape[-1])
    o_ref[0] = jnp.max(v, axis=(1, 3)).reshape(
        (oh // 2) * (ow // 2), y.shape[-1]).astype(o_ref.dtype)


def _conv1_pool(xr, w1r, b1, sc, sh, *, oh, ow, kh, kwid, cout):
    n, rows, _ = xr.shape
    m2 = (oh // 2) * (ow // 2)
    out = pl.pallas_call(
        functools.partial(_conv1_pool_kernel, oh=oh, ow=ow, kh=kh),
        out_shape=jax.ShapeDtypeStruct((n, m2, cout), jnp.bfloat16),
        grid=(n,),
        in_specs=[pl.BlockSpec((1, rows, kwid), lambda i: (i, 0, 0)),
                  pl.BlockSpec((kh, kwid, cout), lambda i: (0, 0, 0)),
                  pl.BlockSpec((1, cout), lambda i: (0, 0)),
                  pl.BlockSpec((1, cout), lambda i: (0, 0)),
                  pl.BlockSpec((1, cout), lambda i: (0, 0))],
        out_specs=pl.BlockSpec((1, m2, cout), lambda i: (i, 0, 0)),
        compiler_params=pltpu.CompilerParams(
            dimension_semantics=("parallel",),
            vmem_limit_bytes=VMEM_LIMIT),
    )(xr, w1r, b1, sc, sh)
    return out


def _conv2_kernel(x_ref, w_ref, b_ref, z_ref, s_ref, q_ref, *, oh, ow, kh):
    x = x_ref[0]
    m = oh * ow
    z = None
    for a in range(kh):
        pa = jnp.concatenate(
            [x[a:a + oh, b:b + ow, :] for b in range(kh)], axis=-1)
        d = jnp.dot(pa.reshape(m, -1), w_ref[a],
                    preferred_element_type=jnp.float32)
        z = d if z is None else z + d
    z = z + b_ref[...]
    z_ref[0] = z.astype(z_ref.dtype)
    s_ref[0] = jnp.sum(z, axis=0, keepdims=True)
    q_ref[0] = jnp.sum(z * z, axis=0, keepdims=True)


def _conv2(xs2, w2r, b2, *, oh, ow, kh, cout):
    n, ih, iw, cin4 = xs2.shape
    m = oh * ow
    z, s, q = pl.pallas_call(
        functools.partial(_conv2_kernel, oh=oh, ow=ow, kh=kh),
        out_shape=(jax.ShapeDtypeStruct((n, m, cout), jnp.float32),
                   jax.ShapeDtypeStruct((n, 1, cout), jnp.float32),
                   jax.ShapeDtypeStruct((n, 1, cout), jnp.float32)),
        grid=(n,),
        in_specs=[pl.BlockSpec((1, ih, iw, cin4), lambda i: (i, 0, 0, 0)),
                  pl.BlockSpec(w2r.shape, lambda i: (0, 0, 0)),
                  pl.BlockSpec((1, cout), lambda i: (0, 0))],
        out_specs=(pl.BlockSpec((1, m, cout), lambda i: (i, 0, 0)),
                   pl.BlockSpec((1, 1, cout), lambda i: (i, 0, 0)),
                   pl.BlockSpec((1, 1, cout), lambda i: (i, 0, 0))),
        compiler_params=pltpu.CompilerParams(
            dimension_semantics=("parallel",),
            vmem_limit_bytes=VMEM_LIMIT),
    )(xs2, w2r, b2)
    return z, s, q


def _pool_kernel(z_ref, sc_ref, sh_ref, o_ref, *, c):
    y = _lrelu(z_ref[...].astype(jnp.float32) * sc_ref[...] + sh_ref[...])
    m = jnp.maximum(y[:, 0], y[:, 1])
    o_ref[...] = jnp.maximum(m[..., :c], m[..., c:]).astype(o_ref.dtype)


def _pool(z4, sc, sh, *, c, cap):
    rows, _, ow2, _ = z4.shape
    hb = _largest_divisor_leq(rows, cap)
    out = pl.pallas_call(
        functools.partial(_pool_kernel, c=c),
        out_shape=jax.ShapeDtypeStruct((rows, ow2, c), jnp.bfloat16),
        grid=(rows // hb,),
        in_specs=[pl.BlockSpec((hb, 2, ow2, 2 * c), lambda t: (t, 0, 0, 0)),
                  pl.BlockSpec((1, 2 * c), lambda t: (0, 0)),
                  pl.BlockSpec((1, 2 * c), lambda t: (0, 0))],
        out_specs=pl.BlockSpec((hb, ow2, c), lambda t: (t, 0, 0)),
        compiler_params=pltpu.CompilerParams(
            dimension_semantics=("parallel",),
            vmem_limit_bytes=VMEM_LIMIT),
    )(z4, sc, sh)
    return out


def _stats_to_affine(s, q, cnt, gamma, beta):
    mu = s / cnt
    var = jnp.maximum(q / cnt - mu * mu, 0.0)
    sc = gamma * jax.lax.rsqrt(var + EPS)
    sh = beta - mu * sc
    return sc, sh


def _dup(v):
    return jnp.concatenate([v, v]).reshape(1, -1)


def _im2col(x, k, stride, pad):
    x = jnp.pad(x, ((0, 0), (pad, pad), (pad, pad), (0, 0)))
    n, hp, wp, c = x.shape
    oh = (hp - k) // stride + 1
    ow = (wp - k) // stride + 1
    rows = []
    for i in range(k):
        cols = []
        for j in range(k):
            cols.append(x[:, i:i + (oh - 1) * stride + 1:stride,
                             j:j + (ow - 1) * stride + 1:stride, :])
        rows.append(jnp.stack(cols, axis=3))
    pch = jnp.stack(rows, axis=3)
    return pch.reshape(n * oh * ow, k * k * c)


def _bn_rows(x, g, b, cnt):
    mu = jnp.sum(x, axis=0, keepdims=True) / cnt
    var = jnp.maximum(jnp.sum(x * x, axis=0, keepdims=True) / cnt - mu * mu,
                      0.0)
    return _lrelu((x - mu) * jax.lax.rsqrt(var + EPS) * g + b)


def _tail_kernel(x_ref, w3_ref, b3_ref, g3_ref, be3_ref,
                 w4_ref, b4_ref, g4_ref, be4_ref,
                 w5_ref, b5_ref, g5_ref, be5_ref,
                 w6_ref, b6_ref, o_ref, *, n, k3):
    x3 = x_ref[...]
    zs = []
    for i in (0, 1):
        for j in (0, 1):
            acc = None
            for kh in range(4):
                r = 2 * i - 1 + kh
                if r < 0 or r > 3:
                    continue
                for kw in range(4):
                    c = 2 * j - 1 + kw
                    if c < 0 or c > 3:
                        continue
                    t = jnp.dot(x3[:, r * 4 + c, :], w3_ref[kh * 4 + kw],
                                preferred_element_type=jnp.float32)
                    acc = t if acc is None else acc + t
            zs.append(acc + b3_ref[...])

    cnt = jnp.float32(4 * n)
    zsum = zs[0] + zs[1] + zs[2] + zs[3]
    qsum = zs[0] * zs[0] + zs[1] * zs[1] + zs[2] * zs[2] + zs[3] * zs[3]
    mu = jnp.sum(zsum, axis=0, keepdims=True) / cnt
    var = jnp.maximum(jnp.sum(qsum, axis=0, keepdims=True) / cnt - mu * mu,
                      0.0)
    sc = g3_ref[...] * jax.lax.rsqrt(var + EPS)
    sh = be3_ref[...] - mu * sc
    y = [_lrelu(z * sc + sh) for z in zs]
    x = jnp.maximum(jnp.maximum(y[0], y[1]), jnp.maximum(y[2], y[3]))

    x = jnp.dot(x.astype(jnp.bfloat16), w4_ref[...],
                preferred_element_type=jnp.float32) + b4_ref[...]
    x = _bn_rows(x, g4_ref[...], be4_ref[...], jnp.float32(n))
    x = jnp.dot(x.astype(jnp.bfloat16), w5_ref[...],
                preferred_element_type=jnp.float32) + b5_ref[...]
    x = _bn_rows(x, g5_ref[...], be5_ref[...], jnp.float32(n))
    o_ref[...] = jnp.dot(x.astype(jnp.bfloat16), w6_ref[...],
                         preferred_element_type=jnp.float32) + b6_ref[...]


def _tail(x3, w3r, p, *, n):
    vm = pl.BlockSpec(memory_space=pltpu.MemorySpace.VMEM)
    return pl.pallas_call(
        functools.partial(_tail_kernel, n=n, k3=0),
        out_shape=jax.ShapeDtypeStruct((n, p["f3_w"].shape[1]), jnp.float32),
        in_specs=[vm] * 15,
        out_specs=vm,
        compiler_params=pltpu.CompilerParams(vmem_limit_bytes=VMEM_LIMIT),
    )(x3, w3r, p["c3_b"], p["c3_g"], p["c3_be"],
      p["f1_w"], p["f1_b"], p["f1_g"], p["f1_be"],
      p["f2_w"], p["f2_b"], p["f2_g"], p["f2_be"],
      p["f3_w"], p["f3_b"])


def _forward(observation, p):
    n = observation.shape[0]
    x = observation.astype(jnp.bfloat16)

    xp = jnp.pad(x, ((0, 0), (1, 1), (1, 1), (0, 0)))
    xs = xp.reshape(n, 49, 2, 49, 2, 3).transpose(0, 1, 3, 2, 4, 5)
    xs = xs.reshape(n, 49, 49, 12)
    xr = jnp.stack([xs[:, :, b:b + 42, :] for b in range(8)], axis=3)
    xr = xr.reshape(n, 49 * 42, 96)
    w1r = p["c1_w"].reshape(8, 2, 8, 2, 3, 128)
    w1r = w1r.transpose(0, 2, 1, 3, 4, 5).reshape(8, 96, 128)
    s1, q1 = _conv1_stats(xr, w1r, p["c1_b"], oh=42, ow=42, kh=8,
                          kwid=96, cout=128)
    sc1, sh1 = _stats_to_affine(s1.reshape(n, 128).sum(0),
                                q1.reshape(n, 128).sum(0),
                                jnp.float32(n * 1764), p["c1_g"], p["c1_be"])
    x2 = _conv1_pool(xr, w1r, p["c1_b"], sc1.reshape(1, 128),
                     sh1.reshape(1, 128), oh=42, ow=42, kh=8,
                     kwid=96, cout=128)
    x2 = x2.reshape(n, 21, 21, 128)

    xp2 = jnp.pad(x2, ((0, 0), (1, 2), (1, 2), (0, 0)))
    xs2 = xp2.reshape(n, 12, 2, 12, 2, 128).transpose(0, 1, 3, 2, 4, 5)
    xs2 = xs2.reshape(n, 12, 12, 512)
    w2r = p["c2_w"].reshape(4, 2, 4, 2, 128, 128)
    w2r = w2r.transpose(0, 2, 1, 3, 4, 5).reshape(4, 2048, 128)
    z2, s2, q2 = _conv2(xs2, w2r, p["c2_b"], oh=8, ow=8, kh=4, cout=128)
    sc2, sh2 = _stats_to_affine(s2.reshape(n, 128).sum(0),
                                q2.reshape(n, 128).sum(0),
                                jnp.float32(n * 64), p["c2_g"], p["c2_be"])
    z24 = z2.reshape(n * 4, 2, 4, 256)
    x3 = _pool(z24, _dup(sc2), _dup(sh2), c=128, cap=512)
    x3 = x3.reshape(n, 4, 4, 128)[..., :64].reshape(n, 16, 64)

    w3r = p["c3_w"].reshape(16, 64, 32)
    logits = _tail(x3, w3r, p, n=n)
    return logits[:, :NCLS]


def kernel(observation,
           c1_w, c1_b, c1_g, c1_be,
           c2_w, c2_b, c2_g, c2_be,
           c3_w, c3_b, c3_g, c3_be,
           f1_w, f1_b, f1_g, f1_be,
           f2_w, f2_b, f2_g, f2_be,
           f3_w, f3_b):
    p = {
        "c1_w": c1_w, "c1_b": c1_b, "c1_g": c1_g, "c1_be": c1_be,
        "c2_w": c2_w, "c2_b": c2_b, "c2_g": c2_g, "c2_be": c2_be,
        "c3_w": c3_w, "c3_b": c3_b, "c3_g": c3_g, "c3_be": c3_be,
        "f1_w": f1_w, "f1_b": f1_b, "f1_g": f1_g, "f1_be": f1_be,
        "f2_w": f2_w, "f2_b": f2_b, "f2_g": f2_g, "f2_be": f2_be,
        "f3_w": f3_w, "f3_b": f3_b,
    }
    return _forward(observation, p)


# speedup vs baseline: 1.2353x; 1.2353x over previous
import functools

import jax
import jax.numpy as jnp
from jax.experimental import pallas as pl
from jax.experimental.pallas import tpu as pltpu

EPS = 1e-5
SLOPE = 0.01
NCLS = 9
VMEM_LIMIT = 64 * 1024 * 1024


def _lrelu(y):
    return jnp.where(y >= 0.0, y, SLOPE * y)


def _largest_divisor_leq(v, cap):
    for d in range(min(cap, v), 0, -1):
        if v % d == 0:
            return d
    return 1


def _conv1_kernel(x_ref, w_ref, b_ref, z_ref, s_ref, q_ref, *, oh, ow, kh):
    x = x_ref[0]
    m = oh * ow
    z = None
    for a in range(kh):
        pa = x[a * ow: a * ow + m, :]
        d = jnp.dot(pa, w_ref[a], preferred_element_type=jnp.float32)
        z = d if z is None else z + d
    z = z + b_ref[...]
    z_ref[0] = z.astype(z_ref.dtype)
    s_ref[0] = jnp.sum(z, axis=0, keepdims=True)
    q_ref[0] = jnp.sum(z * z, axis=0, keepdims=True)


def _conv1(xr, w1r, b1, *, oh, ow, kh, kwid, cout):
    n, rows, _ = xr.shape
    m = oh * ow
    z, s, q = pl.pallas_call(
        functools.partial(_conv1_kernel, oh=oh, ow=ow, kh=kh),
        out_shape=(jax.ShapeDtypeStruct((n, m, cout), jnp.float32),
                   jax.ShapeDtypeStruct((n, 1, cout), jnp.float32),
                   jax.ShapeDtypeStruct((n, 1, cout), jnp.float32)),
        grid=(n,),
        in_specs=[pl.BlockSpec((1, rows, kwid), lambda i: (i, 0, 0)),
                  pl.BlockSpec((kh, kwid, cout), lambda i: (0, 0, 0)),
                  pl.BlockSpec((1, cout), lambda i: (0, 0))],
        out_specs=(pl.BlockSpec((1, m, cout), lambda i: (i, 0, 0)),
                   pl.BlockSpec((1, 1, cout), lambda i: (i, 0, 0)),
                   pl.BlockSpec((1, 1, cout), lambda i: (i, 0, 0))),
        compiler_params=pltpu.CompilerParams(
            dimension_semantics=("parallel",),
            vmem_limit_bytes=VMEM_LIMIT),
    )(xr, w1r, b1)
    return z, s, q


def _conv2_kernel(x_ref, w_ref, b_ref, z_ref, s_ref, q_ref, *, oh, ow, kh):
    x = x_ref[0]
    m = oh * ow
    z = None
    for a in range(kh):
        pa = jnp.concatenate(
            [x[a:a + oh, b:b + ow, :] for b in range(kh)], axis=-1)
        d = jnp.dot(pa.reshape(m, -1), w_ref[a],
                    preferred_element_type=jnp.float32)
        z = d if z is None else z + d
    z = z + b_ref[...]
    z_ref[0] = z.astype(z_ref.dtype)
    s_ref[0] = jnp.sum(z, axis=0, keepdims=True)
    q_ref[0] = jnp.sum(z * z, axis=0, keepdims=True)


def _conv2(xs2, w2r, b2, *, oh, ow, kh, cout):
    n, ih, iw, cin4 = xs2.shape
    m = oh * ow
    z, s, q = pl.pallas_call(
        functools.partial(_conv2_kernel, oh=oh, ow=ow, kh=kh),
        out_shape=(jax.ShapeDtypeStruct((n, m, cout), jnp.float32),
                   jax.ShapeDtypeStruct((n, 1, cout), jnp.float32),
                   jax.ShapeDtypeStruct((n, 1, cout), jnp.float32)),
        grid=(n,),
        in_specs=[pl.BlockSpec((1, ih, iw, cin4), lambda i: (i, 0, 0, 0)),
                  pl.BlockSpec(w2r.shape, lambda i: (0, 0, 0)),
                  pl.BlockSpec((1, cout), lambda i: (0, 0))],
        out_specs=(pl.BlockSpec((1, m, cout), lambda i: (i, 0, 0)),
                   pl.BlockSpec((1, 1, cout), lambda i: (i, 0, 0)),
                   pl.BlockSpec((1, 1, cout), lambda i: (i, 0, 0))),
        compiler_params=pltpu.CompilerParams(
            dimension_semantics=("parallel",),
            vmem_limit_bytes=VMEM_LIMIT),
    )(xs2, w2r, b2)
    return z, s, q


def _pool_kernel(z_ref, sc_ref, sh_ref, o_ref, *, c):
    y = _lrelu(z_ref[...].astype(jnp.float32) * sc_ref[...] + sh_ref[...])
    m = jnp.maximum(y[:, 0], y[:, 1])
    o_ref[...] = jnp.maximum(m[..., :c], m[..., c:]).astype(o_ref.dtype)


def _pool(z4, sc, sh, *, c, cap):
    rows, _, ow2, _ = z4.shape
    hb = _largest_divisor_leq(rows, cap)
    out = pl.pallas_call(
        functools.partial(_pool_kernel, c=c),
        out_shape=jax.ShapeDtypeStruct((rows, ow2, c), jnp.bfloat16),
        grid=(rows // hb,),
        in_specs=[pl.BlockSpec((hb, 2, ow2, 2 * c), lambda t: (t, 0, 0, 0)),
                  pl.BlockSpec((1, 2 * c), lambda t: (0, 0)),
                  pl.BlockSpec((1, 2 * c), lambda t: (0, 0))],
        out_specs=pl.BlockSpec((hb, ow2, c), lambda t: (t, 0, 0)),
        compiler_params=pltpu.CompilerParams(
            dimension_semantics=("parallel",),
            vmem_limit_bytes=VMEM_LIMIT),
    )(z4, sc, sh)
    return out


def _stats_to_affine(s, q, cnt, gamma, beta):
    mu = s / cnt
    var = jnp.maximum(q / cnt - mu * mu, 0.0)
    sc = gamma * jax.lax.rsqrt(var + EPS)
    sh = beta - mu * sc
    return sc, sh


def _dup(v):
    return jnp.concatenate([v, v]).reshape(1, -1)


def _bn_rows(x, g, b, cnt):
    mu = jnp.sum(x, axis=0, keepdims=True) / cnt
    var = jnp.maximum(jnp.sum(x * x, axis=0, keepdims=True) / cnt - mu * mu,
                      0.0)
    return _lrelu((x - mu) * jax.lax.rsqrt(var + EPS) * g + b)


def _tail_kernel(x_ref, w3_ref, b3_ref, g3_ref, be3_ref,
                 w4_ref, b4_ref, g4_ref, be4_ref,
                 w5_ref, b5_ref, g5_ref, be5_ref,
                 w6_ref, b6_ref, o_ref, *, n):
    x3 = x_ref[...]
    zs = []
    for i in (0, 1):
        for j in (0, 1):
            acc = None
            for kh in range(4):
                r = 2 * i - 1 + kh
                if r < 0 or r > 3:
                    continue
                for kw in range(4):
                    c = 2 * j - 1 + kw
                    if c < 0 or c > 3:
                        continue
                    t = jnp.dot(x3[:, r * 4 + c, :], w3_ref[kh * 4 + kw],
                                preferred_element_type=jnp.float32)
                    acc = t if acc is None else acc + t
            zs.append(acc + b3_ref[...])

    cnt = jnp.float32(4 * n)
    zsum = zs[0] + zs[1] + zs[2] + zs[3]
    qsum = zs[0] * zs[0] + zs[1] * zs[1] + zs[2] * zs[2] + zs[3] * zs[3]
    mu = jnp.sum(zsum, axis=0, keepdims=True) / cnt
    var = jnp.maximum(jnp.sum(qsum, axis=0, keepdims=True) / cnt - mu * mu,
                      0.0)
    sc = g3_ref[...] * jax.lax.rsqrt(var + EPS)
    sh = be3_ref[...] - mu * sc
    y = [_lrelu(z * sc + sh) for z in zs]
    x = jnp.maximum(jnp.maximum(y[0], y[1]), jnp.maximum(y[2], y[3]))

    x = jnp.dot(x.astype(jnp.bfloat16), w4_ref[...],
                preferred_element_type=jnp.float32) + b4_ref[...]
    x = _bn_rows(x, g4_ref[...], be4_ref[...], jnp.float32(n))
    x = jnp.dot(x.astype(jnp.bfloat16), w5_ref[...],
                preferred_element_type=jnp.float32) + b5_ref[...]
    x = _bn_rows(x, g5_ref[...], be5_ref[...], jnp.float32(n))
    o_ref[...] = jnp.dot(x.astype(jnp.bfloat16), w6_ref[...],
                         preferred_element_type=jnp.float32) + b6_ref[...]


def _tail(x3, w3r, p, *, n):
    vm = pl.BlockSpec(memory_space=pltpu.MemorySpace.VMEM)
    return pl.pallas_call(
        functools.partial(_tail_kernel, n=n),
        out_shape=jax.ShapeDtypeStruct((n, p["f3_w"].shape[1]), jnp.float32),
        in_specs=[vm] * 15,
        out_specs=vm,
        compiler_params=pltpu.CompilerParams(vmem_limit_bytes=VMEM_LIMIT),
    )(x3, w3r, p["c3_b"], p["c3_g"], p["c3_be"],
      p["f1_w"], p["f1_b"], p["f1_g"], p["f1_be"],
      p["f2_w"], p["f2_b"], p["f2_g"], p["f2_be"],
      p["f3_w"], p["f3_b"])


def _forward(observation, p):
    n = observation.shape[0]
    x = observation.astype(jnp.bfloat16)

    xp = jnp.pad(x, ((0, 0), (1, 1), (1, 1), (0, 0)))
    xs = xp.reshape(n, 49, 2, 49, 2, 3).transpose(0, 1, 3, 2, 4, 5)
    xs = xs.reshape(n, 49, 49, 12)
    xr = jnp.stack([xs[:, :, b:b + 42, :] for b in range(8)], axis=3)
    xr = xr.reshape(n, 49 * 42, 96)
    w1r = p["c1_w"].reshape(8, 2, 8, 2, 3, 128)
    w1r = w1r.transpose(0, 2, 1, 3, 4, 5).reshape(8, 96, 128)
    z1, s1, q1 = _conv1(xr, w1r, p["c1_b"], oh=42, ow=42, kh=8,
                        kwid=96, cout=128)
    sc1, sh1 = _stats_to_affine(s1.reshape(n, 128).sum(0),
                                q1.reshape(n, 128).sum(0),
                                jnp.float32(n * 1764), p["c1_g"], p["c1_be"])
    z14 = z1.reshape(n * 21, 2, 21, 256)
    x2 = _pool(z14, _dup(sc1), _dup(sh1), c=128, cap=128)
    x2 = x2.reshape(n, 21, 21, 128)

    xp2 = jnp.pad(x2, ((0, 0), (1, 2), (1, 2), (0, 0)))
    xs2 = xp2.reshape(n, 12, 2, 12, 2, 128).transpose(0, 1, 3, 2, 4, 5)
    xs2 = xs2.reshape(n, 12, 12, 512)
    w2r = p["c2_w"].reshape(4, 2, 4, 2, 128, 128)
    w2r = w2r.transpose(0, 2, 1, 3, 4, 5).reshape(4, 2048, 128)
    z2, s2, q2 = _conv2(xs2, w2r, p["c2_b"], oh=8, ow=8, kh=4, cout=128)
    sc2, sh2 = _stats_to_affine(s2.reshape(n, 128).sum(0),
                                q2.reshape(n, 128).sum(0),
                                jnp.float32(n * 64), p["c2_g"], p["c2_be"])
    z24 = z2.reshape(n * 4, 2, 4, 256)
    x3 = _pool(z24, _dup(sc2), _dup(sh2), c=128, cap=512)
    x3 = x3.reshape(n, 4, 4, 128)[..., :64].reshape(n, 16, 64)

    w3r = p["c3_w"].reshape(16, 64, 32)
    logits = _tail(x3, w3r, p, n=n)
    return logits[:, :NCLS]


def kernel(observation,
           c1_w, c1_b, c1_g, c1_be,
           c2_w, c2_b, c2_g, c2_be,
           c3_w, c3_b, c3_g, c3_be,
           f1_w, f1_b, f1_g, f1_be,
           f2_w, f2_b, f2_g, f2_be,
           f3_w, f3_b):
    p = {
        "c1_w": c1_w, "c1_b": c1_b, "c1_g": c1_g, "c1_be": c1_be,
        "c2_w": c2_w, "c2_b": c2_b, "c2_g": c2_g, "c2_be": c2_be,
        "c3_w": c3_w, "c3_b": c3_b, "c3_g": c3_g, "c3_be": c3_be,
        "f1_w": f1_w, "f1_b": f1_b, "f1_g": f1_g, "f1_be": f1_be,
        "f2_w": f2_w, "f2_b": f2_b, "f2_g": f2_g, "f2_be": f2_be,
        "f3_w": f3_w, "f3_b": f3_b,
    }
    return _forward(observation, p)


# speedup vs baseline: 1.2551x; 1.0160x over previous
import functools

import jax
import jax.numpy as jnp
from jax.experimental import pallas as pl
from jax.experimental.pallas import tpu as pltpu

EPS = 1e-5
SLOPE = 0.01
NCLS = 9
VMEM_LIMIT = 64 * 1024 * 1024


def _lrelu(y):
    return jnp.where(y >= 0.0, y, SLOPE * y)


def _largest_divisor_leq(v, cap):
    for d in range(min(cap, v), 0, -1):
        if v % d == 0:
            return d
    return 1


def _conv1_kernel(x_ref, w_ref, b_ref, z_ref, s_ref, q_ref,
                  *, oh, ow, kh, ipb):
    m = oh * ow
    for img in range(ipb):
        x = x_ref[img]
        z = None
        for a in range(kh):
            pa = x[a * ow: a * ow + m, :]
            d = jnp.dot(pa, w_ref[a], preferred_element_type=jnp.float32)
            z = d if z is None else z + d
        z = z + b_ref[...]
        z_ref[img] = z.astype(z_ref.dtype)
        s_ref[img] = jnp.sum(z, axis=0, keepdims=True)
        q_ref[img] = jnp.sum(z * z, axis=0, keepdims=True)


def _conv1(xr, w1r, b1, *, oh, ow, kh, kwid, cout, ipb):
    n, rows, _ = xr.shape
    m = oh * ow
    z, s, q = pl.pallas_call(
        functools.partial(_conv1_kernel, oh=oh, ow=ow, kh=kh, ipb=ipb),
        out_shape=(jax.ShapeDtypeStruct((n, m, cout), jnp.float32),
                   jax.ShapeDtypeStruct((n, 1, cout), jnp.float32),
                   jax.ShapeDtypeStruct((n, 1, cout), jnp.float32)),
        grid=(n // ipb,),
        in_specs=[pl.BlockSpec((ipb, rows, kwid), lambda i: (i, 0, 0)),
                  pl.BlockSpec((kh, kwid, cout), lambda i: (0, 0, 0)),
                  pl.BlockSpec((1, cout), lambda i: (0, 0))],
        out_specs=(pl.BlockSpec((ipb, m, cout), lambda i: (i, 0, 0)),
                   pl.BlockSpec((ipb, 1, cout), lambda i: (i, 0, 0)),
                   pl.BlockSpec((ipb, 1, cout), lambda i: (i, 0, 0))),
        compiler_params=pltpu.CompilerParams(
            dimension_semantics=("parallel",),
            vmem_limit_bytes=VMEM_LIMIT),
    )(xr, w1r, b1)
    return z, s, q


def _conv2_kernel(x_ref, w_ref, b_ref, z_ref, s_ref, q_ref, *, oh, ow, kh):
    x = x_ref[0]
    m = oh * ow
    z = None
    for a in range(kh):
        pa = jnp.concatenate(
            [x[a:a + oh, b:b + ow, :] for b in range(kh)], axis=-1)
        d = jnp.dot(pa.reshape(m, -1), w_ref[a],
                    preferred_element_type=jnp.float32)
        z = d if z is None else z + d
    z = z + b_ref[...]
    z_ref[0] = z.astype(z_ref.dtype)
    s_ref[0] = jnp.sum(z, axis=0, keepdims=True)
    q_ref[0] = jnp.sum(z * z, axis=0, keepdims=True)


def _conv2(xs2, w2r, b2, *, oh, ow, kh, cout):
    n, ih, iw, cin4 = xs2.shape
    m = oh * ow
    z, s, q = pl.pallas_call(
        functools.partial(_conv2_kernel, oh=oh, ow=ow, kh=kh),
        out_shape=(jax.ShapeDtypeStruct((n, m, cout), jnp.float32),
                   jax.ShapeDtypeStruct((n, 1, cout), jnp.float32),
                   jax.ShapeDtypeStruct((n, 1, cout), jnp.float32)),
        grid=(n,),
        in_specs=[pl.BlockSpec((1, ih, iw, cin4), lambda i: (i, 0, 0, 0)),
                  pl.BlockSpec(w2r.shape, lambda i: (0, 0, 0)),
                  pl.BlockSpec((1, cout), lambda i: (0, 0))],
        out_specs=(pl.BlockSpec((1, m, cout), lambda i: (i, 0, 0)),
                   pl.BlockSpec((1, 1, cout), lambda i: (i, 0, 0)),
                   pl.BlockSpec((1, 1, cout), lambda i: (i, 0, 0))),
        compiler_params=pltpu.CompilerParams(
            dimension_semantics=("parallel",),
            vmem_limit_bytes=VMEM_LIMIT),
    )(xs2, w2r, b2)
    return z, s, q


def _pool_kernel(z_ref, sc_ref, sh_ref, o_ref, *, c):
    y = _lrelu(z_ref[...].astype(jnp.float32) * sc_ref[...] + sh_ref[...])
    m = jnp.maximum(y[:, 0], y[:, 1])
    o_ref[...] = jnp.maximum(m[..., :c], m[..., c:]).astype(o_ref.dtype)


def _pool(z4, sc, sh, *, c, cap):
    rows, _, ow2, _ = z4.shape
    hb = _largest_divisor_leq(rows, cap)
    out = pl.pallas_call(
        functools.partial(_pool_kernel, c=c),
        out_shape=jax.ShapeDtypeStruct((rows, ow2, c), jnp.bfloat16),
        grid=(rows // hb,),
        in_specs=[pl.BlockSpec((hb, 2, ow2, 2 * c), lambda t: (t, 0, 0, 0)),
                  pl.BlockSpec((1, 2 * c), lambda t: (0, 0)),
                  pl.BlockSpec((1, 2 * c), lambda t: (0, 0))],
        out_specs=pl.BlockSpec((hb, ow2, c), lambda t: (t, 0, 0)),
        compiler_params=pltpu.CompilerParams(
            dimension_semantics=("parallel",),
            vmem_limit_bytes=VMEM_LIMIT),
    )(z4, sc, sh)
    return out


def _stats_to_affine(s, q, cnt, gamma, beta):
    mu = s / cnt
    var = jnp.maximum(q / cnt - mu * mu, 0.0)
    sc = gamma * jax.lax.rsqrt(var + EPS)
    sh = beta - mu * sc
    return sc, sh


def _dup(v):
    return jnp.concatenate([v, v]).reshape(1, -1)


def _bn_rows(x, g, b, cnt):
    mu = jnp.sum(x, axis=0, keepdims=True) / cnt
    var = jnp.maximum(jnp.sum(x * x, axis=0, keepdims=True) / cnt - mu * mu,
                      0.0)
    return _lrelu((x - mu) * jax.lax.rsqrt(var + EPS) * g + b)


def _tail_kernel(x_ref, w3_ref, b3_ref, g3_ref, be3_ref,
                 w4_ref, b4_ref, g4_ref, be4_ref,
                 w5_ref, b5_ref, g5_ref, be5_ref,
                 w6_ref, b6_ref, o_ref, *, n):
    x3 = x_ref[...]
    zs = []
    for i in (0, 1):
        for j in (0, 1):
            acc = None
            for kh in range(4):
                r = 2 * i - 1 + kh
                if r < 0 or r > 3:
                    continue
                for kw in range(4):
                    c = 2 * j - 1 + kw
                    if c < 0 or c > 3:
                        continue
                    t = jnp.dot(x3[:, r * 4 + c, :], w3_ref[kh * 4 + kw],
                                preferred_element_type=jnp.float32)
                    acc = t if acc is None else acc + t
            zs.append(acc + b3_ref[...])

    cnt = jnp.float32(4 * n)
    zsum = zs[0] + zs[1] + zs[2] + zs[3]
    qsum = zs[0] * zs[0] + zs[1] * zs[1] + zs[2] * zs[2] + zs[3] * zs[3]
    mu = jnp.sum(zsum, axis=0, keepdims=True) / cnt
    var = jnp.maximum(jnp.sum(qsum, axis=0, keepdims=True) / cnt - mu * mu,
                      0.0)
    sc = g3_ref[...] * jax.lax.rsqrt(var + EPS)
    sh = be3_ref[...] - mu * sc
    y = [_lrelu(z * sc + sh) for z in zs]
    x = jnp.maximum(jnp.maximum(y[0], y[1]), jnp.maximum(y[2], y[3]))

    x = jnp.dot(x.astype(jnp.bfloat16), w4_ref[...],
                preferred_element_type=jnp.float32) + b4_ref[...]
    x = _bn_rows(x, g4_ref[...], be4_ref[...], jnp.float32(n))
    x = jnp.dot(x.astype(jnp.bfloat16), w5_ref[...],
                preferred_element_type=jnp.float32) + b5_ref[...]
    x = _bn_rows(x, g5_ref[...], be5_ref[...], jnp.float32(n))
    o_ref[...] = jnp.dot(x.astype(jnp.bfloat16), w6_ref[...],
                         preferred_element_type=jnp.float32) + b6_ref[...]


def _tail(x3, w3r, p, *, n):
    vm = pl.BlockSpec(memory_space=pltpu.MemorySpace.VMEM)
    return pl.pallas_call(
        functools.partial(_tail_kernel, n=n),
        out_shape=jax.ShapeDtypeStruct((n, p["f3_w"].shape[1]), jnp.float32),
        in_specs=[vm] * 15,
        out_specs=vm,
        compiler_params=pltpu.CompilerParams(vmem_limit_bytes=VMEM_LIMIT),
    )(x3, w3r, p["c3_b"], p["c3_g"], p["c3_be"],
      p["f1_w"], p["f1_b"], p["f1_g"], p["f1_be"],
      p["f2_w"], p["f2_b"], p["f2_g"], p["f2_be"],
      p["f3_w"], p["f3_b"])


def _forward(observation, p):
    n = observation.shape[0]
    x = observation.astype(jnp.bfloat16)

    xp = jnp.pad(x, ((0, 0), (1, 1), (1, 1), (0, 0)))
    xs = xp.reshape(n, 49, 2, 49, 2, 3).transpose(0, 1, 3, 2, 4, 5)
    xs = xs.reshape(n, 49, 49, 12)
    xr = jnp.stack([xs[:, :, b:b + 42, :] for b in range(8)], axis=3)
    xr = xr.reshape(n, 49 * 42, 96)
    w1r = p["c1_w"].reshape(8, 2, 8, 2, 3, 128)
    w1r = w1r.transpose(0, 2, 1, 3, 4, 5).reshape(8, 96, 128)
    z1, s1, q1 = _conv1(xr, w1r, p["c1_b"], oh=42, ow=42, kh=8,
                        kwid=96, cout=128, ipb=2 if n % 2 == 0 else 1)
    sc1, sh1 = _stats_to_affine(s1.reshape(n, 128).sum(0),
                                q1.reshape(n, 128).sum(0),
                                jnp.float32(n * 1764), p["c1_g"], p["c1_be"])
    z14 = z1.reshape(n * 21, 2, 21, 256)
    x2 = _pool(z14, _dup(sc1), _dup(sh1), c=128, cap=128)
    x2 = x2.reshape(n, 21, 21, 128)

    xp2 = jnp.pad(x2, ((0, 0), (1, 2), (1, 2), (0, 0)))
    xs2 = xp2.reshape(n, 12, 2, 12, 2, 128).transpose(0, 1, 3, 2, 4, 5)
    xs2 = xs2.reshape(n, 12, 12, 512)
    w2r = p["c2_w"].reshape(4, 2, 4, 2, 128, 128)
    w2r = w2r.transpose(0, 2, 1, 3, 4, 5).reshape(4, 2048, 128)
    z2, s2, q2 = _conv2(xs2, w2r, p["c2_b"], oh=8, ow=8, kh=4, cout=128)
    sc2, sh2 = _stats_to_affine(s2.reshape(n, 128).sum(0),
                                q2.reshape(n, 128).sum(0),
                                jnp.float32(n * 64), p["c2_g"], p["c2_be"])
    z24 = z2.reshape(n * 4, 2, 4, 256)
    x3 = _pool(z24, _dup(sc2), _dup(sh2), c=128, cap=512)
    x3 = x3.reshape(n, 4, 4, 128)[..., :64].reshape(n, 16, 64)

    w3r = p["c3_w"].reshape(16, 64, 32)
    logits = _tail(x3, w3r, p, n=n)
    return logits[:, :NCLS]


def kernel(observation,
           c1_w, c1_b, c1_g, c1_be,
           c2_w, c2_b, c2_g, c2_be,
           c3_w, c3_b, c3_g, c3_be,
           f1_w, f1_b, f1_g, f1_be,
           f2_w, f2_b, f2_g, f2_be,
           f3_w, f3_b):
    p = {
        "c1_w": c1_w, "c1_b": c1_b, "c1_g": c1_g, "c1_be": c1_be,
        "c2_w": c2_w, "c2_b": c2_b, "c2_g": c2_g, "c2_be": c2_be,
        "c3_w": c3_w, "c3_b": c3_b, "c3_g": c3_g, "c3_be": c3_be,
        "f1_w": f1_w, "f1_b": f1_b, "f1_g": f1_g, "f1_be": f1_be,
        "f2_w": f2_w, "f2_b": f2_b, "f2_g": f2_g, "f2_be": f2_be,
        "f3_w": f3_w, "f3_b": f3_b,
    }
    return _forward(observation, p)


# speedup vs baseline: 1.2605x; 1.0043x over previous
import functools

import jax
import jax.numpy as jnp
from jax.experimental import pallas as pl
from jax.experimental.pallas import tpu as pltpu

EPS = 1e-5
SLOPE = 0.01
NCLS = 9
VMEM_LIMIT = 64 * 1024 * 1024


def _lrelu(y):
    return jnp.where(y >= 0.0, y, SLOPE * y)


def _largest_divisor_leq(v, cap):
    for d in range(min(cap, v), 0, -1):
        if v % d == 0:
            return d
    return 1


def _conv1_kernel(x_ref, w_ref, b_ref, z_ref, s_ref, q_ref,
                  *, oh, ow, kh, ipb):
    m = oh * ow
    for img in range(ipb):
        x = x_ref[img]
        z = None
        for a in range(kh):
            pa = x[a * ow: a * ow + m, :]
            d = jnp.dot(pa, w_ref[a], preferred_element_type=jnp.float32)
            z = d if z is None else z + d
        z = z + b_ref[...]
        z_ref[img] = z.astype(z_ref.dtype)
        s_ref[img] = jnp.sum(z, axis=0, keepdims=True)
        q_ref[img] = jnp.sum(z * z, axis=0, keepdims=True)


def _conv1(xr, w1r, b1, *, oh, ow, kh, kwid, cout, ipb):
    n, rows, _ = xr.shape
    m = oh * ow
    z, s, q = pl.pallas_call(
        functools.partial(_conv1_kernel, oh=oh, ow=ow, kh=kh, ipb=ipb),
        out_shape=(jax.ShapeDtypeStruct((n, m, cout), jnp.float32),
                   jax.ShapeDtypeStruct((n, 1, cout), jnp.float32),
                   jax.ShapeDtypeStruct((n, 1, cout), jnp.float32)),
        grid=(n // ipb,),
        in_specs=[pl.BlockSpec((ipb, rows, kwid), lambda i: (i, 0, 0)),
                  pl.BlockSpec((kh, kwid, cout), lambda i: (0, 0, 0)),
                  pl.BlockSpec((1, cout), lambda i: (0, 0))],
        out_specs=(pl.BlockSpec((ipb, m, cout), lambda i: (i, 0, 0)),
                   pl.BlockSpec((ipb, 1, cout), lambda i: (i, 0, 0)),
                   pl.BlockSpec((ipb, 1, cout), lambda i: (i, 0, 0))),
        compiler_params=pltpu.CompilerParams(
            dimension_semantics=("parallel",),
            vmem_limit_bytes=VMEM_LIMIT),
    )(xr, w1r, b1)
    return z, s, q


def _conv2_kernel(x_ref, w_ref, b_ref, z_ref, s_ref, q_ref, *, oh, ow, kh):
    x = x_ref[0]
    m = oh * ow
    z = None
    for a in range(kh):
        pa = jnp.concatenate(
            [x[a:a + oh, b:b + ow, :] for b in range(kh)], axis=-1)
        d = jnp.dot(pa.reshape(m, -1), w_ref[a],
                    preferred_element_type=jnp.float32)
        z = d if z is None else z + d
    z = z + b_ref[...]
    z_ref[0] = z.astype(z_ref.dtype)
    s_ref[0] = jnp.sum(z, axis=0, keepdims=True)
    q_ref[0] = jnp.sum(z * z, axis=0, keepdims=True)


def _conv2(xs2, w2r, b2, *, oh, ow, kh, cout):
    n, ih, iw, cin4 = xs2.shape
    m = oh * ow
    z, s, q = pl.pallas_call(
        functools.partial(_conv2_kernel, oh=oh, ow=ow, kh=kh),
        out_shape=(jax.ShapeDtypeStruct((n, m, cout), jnp.float32),
                   jax.ShapeDtypeStruct((n, 1, cout), jnp.float32),
                   jax.ShapeDtypeStruct((n, 1, cout), jnp.float32)),
        grid=(n,),
        in_specs=[pl.BlockSpec((1, ih, iw, cin4), lambda i: (i, 0, 0, 0)),
                  pl.BlockSpec(w2r.shape, lambda i: (0, 0, 0)),
                  pl.BlockSpec((1, cout), lambda i: (0, 0))],
        out_specs=(pl.BlockSpec((1, m, cout), lambda i: (i, 0, 0)),
                   pl.BlockSpec((1, 1, cout), lambda i: (i, 0, 0)),
                   pl.BlockSpec((1, 1, cout), lambda i: (i, 0, 0))),
        compiler_params=pltpu.CompilerParams(
            dimension_semantics=("parallel",),
            vmem_limit_bytes=VMEM_LIMIT),
    )(xs2, w2r, b2)
    return z, s, q


def _pool_kernel(z_ref, sc_ref, sh_ref, o_ref, *, c):
    y = _lrelu(z_ref[...].astype(jnp.float32) * sc_ref[...] + sh_ref[...])
    m = jnp.maximum(y[:, 0], y[:, 1])
    o_ref[...] = jnp.maximum(m[..., :c], m[..., c:]).astype(o_ref.dtype)


def _pool(z4, sc, sh, *, c, cap):
    rows, _, ow2, _ = z4.shape
    hb = _largest_divisor_leq(rows, cap)
    out = pl.pallas_call(
        functools.partial(_pool_kernel, c=c),
        out_shape=jax.ShapeDtypeStruct((rows, ow2, c), jnp.bfloat16),
        grid=(rows // hb,),
        in_specs=[pl.BlockSpec((hb, 2, ow2, 2 * c), lambda t: (t, 0, 0, 0)),
                  pl.BlockSpec((1, 2 * c), lambda t: (0, 0)),
                  pl.BlockSpec((1, 2 * c), lambda t: (0, 0))],
        out_specs=pl.BlockSpec((hb, ow2, c), lambda t: (t, 0, 0)),
        compiler_params=pltpu.CompilerParams(
            dimension_semantics=("parallel",),
            vmem_limit_bytes=VMEM_LIMIT),
    )(z4, sc, sh)
    return out


def _stats_to_affine(s, q, cnt, gamma, beta):
    mu = s / cnt
    var = jnp.maximum(q / cnt - mu * mu, 0.0)
    sc = gamma * jax.lax.rsqrt(var + EPS)
    sh = beta - mu * sc
    return sc, sh


def _dup(v):
    return jnp.concatenate([v, v]).reshape(1, -1)


def _bn_rows(x, g, b, cnt):
    mu = jnp.sum(x, axis=0, keepdims=True) / cnt
    var = jnp.maximum(jnp.sum(x * x, axis=0, keepdims=True) / cnt - mu * mu,
                      0.0)
    return _lrelu((x - mu) * jax.lax.rsqrt(var + EPS) * g + b)


def _tail_kernel(x_ref, w3_ref, b3_ref, g3_ref, be3_ref,
                 w4_ref, b4_ref, g4_ref, be4_ref,
                 w5_ref, b5_ref, g5_ref, be5_ref,
                 w6_ref, b6_ref, o_ref, *, n):
    x3 = x_ref[...]
    zs = []
    for i in (0, 1):
        for j in (0, 1):
            acc = None
            for kh in range(4):
                r = 2 * i - 1 + kh
                if r < 0 or r > 3:
                    continue
                for kw in range(4):
                    c = 2 * j - 1 + kw
                    if c < 0 or c > 3:
                        continue
                    t = jnp.dot(x3[:, r * 4 + c, :], w3_ref[kh * 4 + kw],
                                preferred_element_type=jnp.float32)
                    acc = t if acc is None else acc + t
            zs.append(acc + b3_ref[...])

    cnt = jnp.float32(4 * n)
    zsum = zs[0] + zs[1] + zs[2] + zs[3]
    qsum = zs[0] * zs[0] + zs[1] * zs[1] + zs[2] * zs[2] + zs[3] * zs[3]
    mu = jnp.sum(zsum, axis=0, keepdims=True) / cnt
    var = jnp.maximum(jnp.sum(qsum, axis=0, keepdims=True) / cnt - mu * mu,
                      0.0)
    sc = g3_ref[...] * jax.lax.rsqrt(var + EPS)
    sh = be3_ref[...] - mu * sc
    y = [_lrelu(z * sc + sh) for z in zs]
    x = jnp.maximum(jnp.maximum(y[0], y[1]), jnp.maximum(y[2], y[3]))

    x = jnp.dot(x.astype(jnp.bfloat16), w4_ref[...],
                preferred_element_type=jnp.float32) + b4_ref[...]
    x = _bn_rows(x, g4_ref[...], be4_ref[...], jnp.float32(n))
    x = jnp.dot(x.astype(jnp.bfloat16), w5_ref[...],
                preferred_element_type=jnp.float32) + b5_ref[...]
    x = _bn_rows(x, g5_ref[...], be5_ref[...], jnp.float32(n))
    o_ref[...] = jnp.dot(x.astype(jnp.bfloat16), w6_ref[...],
                         preferred_element_type=jnp.float32) + b6_ref[...]


def _tail(x3, w3r, p, *, n):
    vm = pl.BlockSpec(memory_space=pltpu.MemorySpace.VMEM)
    return pl.pallas_call(
        functools.partial(_tail_kernel, n=n),
        out_shape=jax.ShapeDtypeStruct((n, p["f3_w"].shape[1]), jnp.float32),
        in_specs=[vm] * 15,
        out_specs=vm,
        compiler_params=pltpu.CompilerParams(vmem_limit_bytes=VMEM_LIMIT),
    )(x3, w3r, p["c3_b"], p["c3_g"], p["c3_be"],
      p["f1_w"], p["f1_b"], p["f1_g"], p["f1_be"],
      p["f2_w"], p["f2_b"], p["f2_g"], p["f2_be"],
      p["f3_w"], p["f3_b"])


def _forward(observation, p):
    n = observation.shape[0]
    x = observation.astype(jnp.bfloat16)

    xp = jnp.pad(x, ((0, 0), (1, 1), (1, 1), (0, 0)))
    xs = xp.reshape(n, 49, 2, 49, 2, 3).transpose(0, 1, 3, 2, 4, 5)
    xs = xs.reshape(n, 49, 49, 12)
    xr = jnp.stack([xs[:, :, b:b + 42, :] for b in range(8)], axis=3)
    xr = xr.reshape(n, 49 * 42, 96)
    w1r = p["c1_w"].reshape(8, 2, 8, 2, 3, 128)
    w1r = w1r.transpose(0, 2, 1, 3, 4, 5).reshape(8, 96, 128)
    z1, s1, q1 = _conv1(xr, w1r, p["c1_b"], oh=42, ow=42, kh=8,
                        kwid=96, cout=128, ipb=4 if n % 4 == 0 else 1)
    sc1, sh1 = _stats_to_affine(s1.reshape(n, 128).sum(0),
                                q1.reshape(n, 128).sum(0),
                                jnp.float32(n * 1764), p["c1_g"], p["c1_be"])
    z14 = z1.reshape(n * 21, 2, 21, 256)
    x2 = _pool(z14, _dup(sc1), _dup(sh1), c=128, cap=128)
    x2 = x2.reshape(n, 21, 21, 128)

    xp2 = jnp.pad(x2, ((0, 0), (1, 2), (1, 2), (0, 0)))
    xs2 = xp2.reshape(n, 12, 2, 12, 2, 128).transpose(0, 1, 3, 2, 4, 5)
    xs2 = xs2.reshape(n, 12, 12, 512)
    w2r = p["c2_w"].reshape(4, 2, 4, 2, 128, 128)
    w2r = w2r.transpose(0, 2, 1, 3, 4, 5).reshape(4, 2048, 128)
    z2, s2, q2 = _conv2(xs2, w2r, p["c2_b"], oh=8, ow=8, kh=4, cout=128)
    sc2, sh2 = _stats_to_affine(s2.reshape(n, 128).sum(0),
                                q2.reshape(n, 128).sum(0),
                                jnp.float32(n * 64), p["c2_g"], p["c2_be"])
    z24 = z2.reshape(n * 4, 2, 4, 256)
    x3 = _pool(z24, _dup(sc2), _dup(sh2), c=128, cap=512)
    x3 = x3.reshape(n, 4, 4, 128)[..., :64].reshape(n, 16, 64)

    w3r = p["c3_w"].reshape(16, 64, 32)
    logits = _tail(x3, w3r, p, n=n)
    return logits[:, :NCLS]


def kernel(observation,
           c1_w, c1_b, c1_g, c1_be,
           c2_w, c2_b, c2_g, c2_be,
           c3_w, c3_b, c3_g, c3_be,
           f1_w, f1_b, f1_g, f1_be,
           f2_w, f2_b, f2_g, f2_be,
           f3_w, f3_b):
    p = {
        "c1_w": c1_w, "c1_b": c1_b, "c1_g": c1_g, "c1_be": c1_be,
        "c2_w": c2_w, "c2_b": c2_b, "c2_g": c2_g, "c2_be": c2_be,
        "c3_w": c3_w, "c3_b": c3_b, "c3_g": c3_g, "c3_be": c3_be,
        "f1_w": f1_w, "f1_b": f1_b, "f1_g": f1_g, "f1_be": f1_be,
        "f2_w": f2_w, "f2_b": f2_b, "f2_g": f2_g, "f2_be": f2_be,
        "f3_w": f3_w, "f3_b": f3_b,
    }
    return _forward(observation, p)
